# Initial kernel scaffold; baseline (speedup 1.0000x reference)
import jax
import jax.numpy as jnp
from jax.experimental import pallas as pl


def kernel(cluster, q, Wq, Wkv, Wp, bp):
    raise NotImplementedError("write your pallas kernel here")


# speedup vs baseline: 4.1516x; 4.1516x over previous
import jax
import jax.numpy as jnp
from jax.experimental import pallas as pl

B, N, D = 4, 2048, 1024
H = 16
C = 128
DH = D // H
NB = 512


def _proj_segsum_body(cluster_ref, q_ref, wqt_ref, wkvt_ref,
                      qp_ref, kvsum_ref, counts_ref):
    b = pl.program_id(0)
    i = pl.program_id(1)
    x = q_ref[0]
    qp_ref[0] = jax.lax.dot_general(
        x, wqt_ref[...], (((1,), (0,)), ((), ())),
        preferred_element_type=jnp.float32)
    kv = jax.lax.dot_general(
        x, wkvt_ref[...], (((1,), (0,)), ((), ())),
        preferred_element_type=jnp.float32)
    ids_all = cluster_ref[0]
    iota_c = jax.lax.broadcasted_iota(jnp.int32, (NB, C), 1)
    oh4 = jnp.concatenate(
        [jnp.where(iota_c == ids_all[bc, :][:, None], 1.0, 0.0)
         for bc in range(B)], axis=1).astype(jnp.float32)

    @pl.when(jnp.logical_and(b == 0, i == 0))
    def _():
        counts_ref[...] = jnp.zeros_like(counts_ref)

    @pl.when(i == 0)
    def _():
        kvsum_ref[...] = jnp.zeros_like(kvsum_ref)

    @pl.when(b == 0)
    def _():
        counts_ref[0] += jnp.sum(oh4, axis=0).reshape(B, C)

    kvsum_ref[0] += jax.lax.dot_general(
        oh4, kv, (((0,), (0,)), ((), ())),
        preferred_element_type=jnp.float32)


def _attn_body(qp_ref, kvsum_ref, counts_ref, wpt_ref, bp_ref, out_ref):
    counts4 = counts_ref[0]
    xs = []
    for h in range(H):
        bc = h % B
        counts = counts4[bc:bc + 1, :]
        w = jnp.where(counts > 0, 1.0 / counts, 0.0) * (1.0 / 8.0)
        qh = qp_ref[0, :, h * DH:(h + 1) * DH]
        ksum = kvsum_ref[0, bc * C:(bc + 1) * C, h * DH:(h + 1) * DH]
        vsum = kvsum_ref[0, bc * C:(bc + 1) * C, D + h * DH:D + (h + 1) * DH]
        s = jax.lax.dot_general(
            qh, ksum, (((1,), (1,)), ((), ())),
            preferred_element_type=jnp.float32) * w
        m = jnp.max(s, axis=-1, keepdims=True)
        e = jnp.exp(s - m)
        z = jnp.sum(e * counts, axis=-1, keepdims=True)
        p = e / z
        xs.append(jax.lax.dot_general(
            p, vsum, (((1,), (0,)), ((), ())),
            preferred_element_type=jnp.float32))
    x = jnp.concatenate(xs, axis=1)
    out_ref[0] = jax.lax.dot_general(
        x, wpt_ref[...], (((1,), (0,)), ((), ())),
        preferred_element_type=jnp.float32) + bp_ref[...]


def kernel(cluster, q, Wq, Wkv, Wp, bp):
    nb = N // NB
    cl_t = cluster.reshape(B, nb, NB).transpose(1, 0, 2)
    qp, kvsum, counts = pl.pallas_call(
        _proj_segsum_body,
        grid=(B, nb),
        in_specs=[
            pl.BlockSpec((1, B, NB), lambda b, i: (i, 0, 0)),
            pl.BlockSpec((1, NB, D), lambda b, i: (b, i, 0)),
            pl.BlockSpec((D, D), lambda b, i: (0, 0)),
            pl.BlockSpec((D, 2 * D), lambda b, i: (0, 0)),
        ],
        out_specs=[
            pl.BlockSpec((1, NB, D), lambda b, i: (b, i, 0)),
            pl.BlockSpec((1, B * C, 2 * D), lambda b, i: (b, 0, 0)),
            pl.BlockSpec((1, B, C), lambda b, i: (0, 0, 0)),
        ],
        out_shape=[
            jax.ShapeDtypeStruct((B, N, D), jnp.float32),
            jax.ShapeDtypeStruct((B, B * C, 2 * D), jnp.float32),
            jax.ShapeDtypeStruct((1, B, C), jnp.float32),
        ],
    )(cl_t, q, Wq.T, Wkv.T)

    out = pl.pallas_call(
        _attn_body,
        grid=(B, nb),
        in_specs=[
            pl.BlockSpec((1, NB, D), lambda b, i: (b, i, 0)),
            pl.BlockSpec((1, B * C, 2 * D), lambda b, i: (b, 0, 0)),
            pl.BlockSpec((1, B, C), lambda b, i: (0, 0, 0)),
            pl.BlockSpec((D, D), lambda b, i: (0, 0)),
            pl.BlockSpec((1, D), lambda b, i: (0, 0)),
        ],
        out_specs=pl.BlockSpec((1, NB, D), lambda b, i: (b, i, 0)),
        out_shape=jax.ShapeDtypeStruct((B, N, D), jnp.float32),
    )(qp, kvsum, counts, Wp.T, bp.reshape(1, D))
    return out


# speedup vs baseline: 4.3444x; 1.0464x over previous
import jax
import jax.numpy as jnp
from jax.experimental import pallas as pl

B, N, D = 4, 2048, 1024
H = 16
C = 128
DH = D // H
NB = 512


def _proj_segsum_body(cluster_ref, q_ref, wqt_ref, wkvt_ref,
                      qp_ref, kvsum_ref, counts_ref):
    b = pl.program_id(0)
    i = pl.program_id(1)
    x = q_ref[0]
    qp_ref[0] = jax.lax.dot_general(
        x, wqt_ref[...], (((1,), (0,)), ((), ())),
        preferred_element_type=jnp.float32).astype(jnp.bfloat16)
    kv = jax.lax.dot_general(
        x, wkvt_ref[...], (((1,), (0,)), ((), ())),
        preferred_element_type=jnp.float32).astype(jnp.bfloat16)
    ids_all = cluster_ref[0]
    iota_c = jax.lax.broadcasted_iota(jnp.int32, (NB, C), 1)
    oh4 = jnp.concatenate(
        [jnp.where(iota_c == ids_all[bc, :][:, None], 1.0, 0.0)
         for bc in range(B)], axis=1).astype(jnp.bfloat16)

    @pl.when(jnp.logical_and(b == 0, i == 0))
    def _():
        counts_ref[...] = jnp.zeros_like(counts_ref)

    @pl.when(i == 0)
    def _():
        kvsum_ref[...] = jnp.zeros_like(kvsum_ref)

    @pl.when(b == 0)
    def _():
        counts_ref[0] += jnp.sum(oh4, axis=0).reshape(B, C)

    kvsum_ref[0] += jax.lax.dot_general(
        oh4, kv, (((0,), (0,)), ((), ())),
        preferred_element_type=jnp.float32)


def _attn_body(qp_ref, kvsum_ref, counts_ref, wpt_ref, bp_ref, out_ref):
    counts4 = counts_ref[0]
    xs = []
    for h in range(H):
        bc = h % B
        counts = counts4[bc:bc + 1, :]
        w = jnp.where(counts > 0, 1.0 / counts, 0.0) * (1.0 / 8.0)
        qh = qp_ref[0, :, h * DH:(h + 1) * DH]
        ksum = kvsum_ref[0, bc * C:(bc + 1) * C, h * DH:(h + 1) * DH]
        vsum = kvsum_ref[0, bc * C:(bc + 1) * C, D + h * DH:D + (h + 1) * DH]
        s = jax.lax.dot_general(
            qh, ksum.astype(jnp.bfloat16), (((1,), (1,)), ((), ())),
            preferred_element_type=jnp.float32) * w
        m = jnp.max(s, axis=-1, keepdims=True)
        e = jnp.exp(s - m)
        z = jnp.sum(e * counts, axis=-1, keepdims=True)
        p = (e / z).astype(jnp.bfloat16)
        xs.append(jax.lax.dot_general(
            p, vsum.astype(jnp.bfloat16), (((1,), (0,)), ((), ())),
            preferred_element_type=jnp.float32))
    x = jnp.concatenate(xs, axis=1).astype(jnp.bfloat16)
    out_ref[0] = jax.lax.dot_general(
        x, wpt_ref[...], (((1,), (0,)), ((), ())),
        preferred_element_type=jnp.float32) + bp_ref[...]


def kernel(cluster, q, Wq, Wkv, Wp, bp):
    nb = N // NB
    cl_t = cluster.reshape(B, nb, NB).transpose(1, 0, 2)
    qp, kvsum, counts = pl.pallas_call(
        _proj_segsum_body,
        grid=(B, nb),
        in_specs=[
            pl.BlockSpec((1, B, NB), lambda b, i: (i, 0, 0)),
            pl.BlockSpec((1, NB, D), lambda b, i: (b, i, 0)),
            pl.BlockSpec((D, D), lambda b, i: (0, 0)),
            pl.BlockSpec((D, 2 * D), lambda b, i: (0, 0)),
        ],
        out_specs=[
            pl.BlockSpec((1, NB, D), lambda b, i: (b, i, 0)),
            pl.BlockSpec((1, B * C, 2 * D), lambda b, i: (b, 0, 0)),
            pl.BlockSpec((1, B, C), lambda b, i: (0, 0, 0)),
        ],
        out_shape=[
            jax.ShapeDtypeStruct((B, N, D), jnp.bfloat16),
            jax.ShapeDtypeStruct((B, B * C, 2 * D), jnp.float32),
            jax.ShapeDtypeStruct((1, B, C), jnp.float32),
        ],
    )(cl_t, q.astype(jnp.bfloat16), Wq.T.astype(jnp.bfloat16),
      Wkv.T.astype(jnp.bfloat16))

    out = pl.pallas_call(
        _attn_body,
        grid=(B, nb),
        in_specs=[
            pl.BlockSpec((1, NB, D), lambda b, i: (b, i, 0)),
            pl.BlockSpec((1, B * C, 2 * D), lambda b, i: (b, 0, 0)),
            pl.BlockSpec((1, B, C), lambda b, i: (0, 0, 0)),
            pl.BlockSpec((D, D), lambda b, i: (0, 0)),
            pl.BlockSpec((1, D), lambda b, i: (0, 0)),
        ],
        out_specs=pl.BlockSpec((1, NB, D), lambda b, i: (b, i, 0)),
        out_shape=jax.ShapeDtypeStruct((B, N, D), jnp.float32),
    )(qp, kvsum, counts, Wp.T.astype(jnp.bfloat16), bp.reshape(1, D))
    return out


# speedup vs baseline: 6.5254x; 1.5020x over previous
import jax
import jax.numpy as jnp
from jax.experimental import pallas as pl
from jax.experimental.pallas import tpu as pltpu

B, N, D = 4, 2048, 1024
H = 16
C = 128
DH = D // H
NB = 512
BC = B * C


def _proj_segsum_body(cluster_ref, q_ref, wq_ref, wkv_ref,
                      qp_ref, kc8_ref, vaug_ref,
                      kvsum_ref, counts_ref):
    b = pl.program_id(0)
    i = pl.program_id(1)
    nb = pl.num_programs(1)
    x = q_ref[0].astype(jnp.bfloat16)
    qp_ref[0] = jax.lax.dot_general(
        x, wq_ref[...], (((1,), (1,)), ((), ())),
        preferred_element_type=jnp.float32).astype(jnp.bfloat16)
    kv = jax.lax.dot_general(
        x, wkv_ref[...], (((1,), (1,)), ((), ())),
        preferred_element_type=jnp.float32).astype(jnp.bfloat16)
    ids_all = cluster_ref[0]
    iota_c = jax.lax.broadcasted_iota(jnp.int32, (NB, C), 1)
    oh4 = jnp.concatenate(
        [jnp.where(iota_c == ids_all[bc, :][:, None], 1.0, 0.0)
         for bc in range(B)], axis=1).astype(jnp.bfloat16)

    @pl.when(jnp.logical_and(b == 0, i == 0))
    def _():
        counts_ref[...] = jnp.zeros_like(counts_ref)

    @pl.when(i == 0)
    def _():
        kvsum_ref[...] = jnp.zeros_like(kvsum_ref)

    @pl.when(b == 0)
    def _():
        counts_ref[...] += jax.lax.dot_general(
            oh4, jnp.ones((NB, 8), jnp.bfloat16), (((0,), (0,)), ((), ())),
            preferred_element_type=jnp.float32)

    kvsum_ref[...] += jax.lax.dot_general(
        oh4, kv, (((0,), (0,)), ((), ())),
        preferred_element_type=jnp.float32)

    @pl.when(i == nb - 1)
    def _():
        counts_col = counts_ref[:, 0:1]
        w8 = jnp.where(counts_col > 0, 0.125 / counts_col, 0.0)
        kc8_ref[0] = (kvsum_ref[:, :D] * w8).astype(jnp.bfloat16)
        vparts = []
        for h in range(H):
            vparts.append(kvsum_ref[:, D + h * DH:D + (h + 1) * DH])
            vparts.append(counts_col)
            vparts.append(jnp.zeros((BC, C - DH - 1), jnp.float32))
        vaug_ref[0] = jnp.concatenate(vparts, axis=1).astype(jnp.bfloat16)


def _attn_body(qp_ref, kc8_ref, vaug_ref, wp_ref, bp_ref, out_ref):
    xs = []
    for h in range(H):
        bc = h % B
        qh = qp_ref[0, :, h * DH:(h + 1) * DH]
        kc8 = kc8_ref[0, bc * C:(bc + 1) * C, h * DH:(h + 1) * DH]
        vaug = vaug_ref[0, bc * C:(bc + 1) * C, h * C:(h + 1) * C]
        s = jax.lax.dot_general(
            qh, kc8, (((1,), (1,)), ((), ())),
            preferred_element_type=jnp.float32)
        e = jnp.exp(s).astype(jnp.bfloat16)
        r = jax.lax.dot_general(
            e, vaug, (((1,), (0,)), ((), ())),
            preferred_element_type=jnp.float32)
        xs.append(r[:, :DH] / r[:, DH:DH + 1])
    x = jnp.concatenate(xs, axis=1).astype(jnp.bfloat16)
    out_ref[0] = jax.lax.dot_general(
        x, wp_ref[...], (((1,), (1,)), ((), ())),
        preferred_element_type=jnp.float32) + bp_ref[...]


def kernel(cluster, q, Wq, Wkv, Wp, bp):
    nb = N // NB
    cl_t = cluster.reshape(B, nb, NB).transpose(1, 0, 2)
    qp, kc8, vaug = pl.pallas_call(
        _proj_segsum_body,
        grid=(B, nb),
        in_specs=[
            pl.BlockSpec((1, B, NB), lambda b, i: (i, 0, 0)),
            pl.BlockSpec((1, NB, D), lambda b, i: (b, i, 0)),
            pl.BlockSpec((D, D), lambda b, i: (0, 0)),
            pl.BlockSpec((2 * D, D), lambda b, i: (0, 0)),
        ],
        out_specs=[
            pl.BlockSpec((1, NB, D), lambda b, i: (b, i, 0)),
            pl.BlockSpec((1, BC, D), lambda b, i: (b, 0, 0)),
            pl.BlockSpec((1, BC, H * C), lambda b, i: (b, 0, 0)),
        ],
        out_shape=[
            jax.ShapeDtypeStruct((B, N, D), jnp.bfloat16),
            jax.ShapeDtypeStruct((B, BC, D), jnp.bfloat16),
            jax.ShapeDtypeStruct((B, BC, H * C), jnp.bfloat16),
        ],
        scratch_shapes=[
            pltpu.VMEM((BC, 2 * D), jnp.float32),
            pltpu.VMEM((BC, 8), jnp.float32),
        ],
    )(cl_t, q, Wq.astype(jnp.bfloat16), Wkv.astype(jnp.bfloat16))

    out = pl.pallas_call(
        _attn_body,
        grid=(B, nb),
        in_specs=[
            pl.BlockSpec((1, NB, D), lambda b, i: (b, i, 0)),
            pl.BlockSpec((1, BC, D), lambda b, i: (b, 0, 0)),
            pl.BlockSpec((1, BC, H * C), lambda b, i: (b, 0, 0)),
            pl.BlockSpec((D, D), lambda b, i: (0, 0)),
            pl.BlockSpec((1, D), lambda b, i: (0, 0)),
        ],
        out_specs=pl.BlockSpec((1, NB, D), lambda b, i: (b, i, 0)),
        out_shape=jax.ShapeDtypeStruct((B, N, D), jnp.float32),
    )(qp, kc8, vaug, Wp.astype(jnp.bfloat16), bp.reshape(1, D))
    return out
